# baseline (speedup 1.0000x reference)
import functools

import jax
import jax.numpy as jnp
from jax import lax
from jax.experimental import pallas as pl
from jax.experimental.pallas import tpu as pltpu
from jax.experimental.pallas import tpu_sc as plsc

N = 10000
E = 320000
C = 128
R = 8
NB = 4
L = 3

NC = 2
NT = 16
BATCH = 128
KB = 160
KC = 40
E_PT = KB * BATCH
E_PAD = NT * E_PT
SEG_ROWS = 81920
STRIPE = SEG_ROWS // NT
NPASS = 8
NBLK = 10
BN = N // NBLK


def _prep_body(dst_ref, et_ref, seg_ref):
    seg_ref[...] = et_ref[...] * N + dst_ref[...]


def _prep_idx(dst2, et2):
    return pl.pallas_call(
        _prep_body,
        out_shape=jax.ShapeDtypeStruct((NT * KB, BATCH), jnp.int32),
    )(dst2, et2)


def _totable_body(x_ref, t_ref):
    for p in range(NPASS):
        t_ref[p] = x_ref[:, 16 * p:16 * p + 16]


def _totable(x):
    return pl.pallas_call(
        _totable_body,
        grid=(NBLK,),
        in_specs=[pl.BlockSpec((BN, C), lambda i: (i, 0))],
        out_specs=pl.BlockSpec((NPASS, BN, 16), lambda i: (0, i, 0)),
        out_shape=jax.ShapeDtypeStruct((NPASS, N, 16), jnp.float32),
    )(x)


def _lin1_body(x_ref, w_ref, b_ref, t_ref, h_ref, hid_ref):
    h = jnp.dot(x_ref[...], w_ref[...], preferred_element_type=jnp.float32)
    h = h + b_ref[...]
    h_ref[...] = h
    hid_ref[...] = t_ref[0, 0] * h


def _lin1(x, w, b2, t2):
    return pl.pallas_call(
        _lin1_body,
        grid=(NBLK,),
        in_specs=[
            pl.BlockSpec((BN, C), lambda i: (i, 0)),
            pl.BlockSpec((C, C), lambda i: (0, 0)),
            pl.BlockSpec((1, C), lambda i: (0, 0)),
            pl.BlockSpec((1, L + 1), lambda i: (0, 0)),
        ],
        out_specs=[
            pl.BlockSpec((BN, C), lambda i: (i, 0)),
            pl.BlockSpec((BN, C), lambda i: (i, 0)),
        ],
        out_shape=[
            jax.ShapeDtypeStruct((N, C), jnp.float32),
            jax.ShapeDtypeStruct((N, C), jnp.float32),
        ],
    )(x, w, b2, t2)


def _lin2_body(x_ref, w_ref, b_ref, o_ref):
    o_ref[...] = jnp.dot(x_ref[...], w_ref[...],
                         preferred_element_type=jnp.float32) + b_ref[...]


def _lin2(x, w, b2):
    return pl.pallas_call(
        _lin2_body,
        grid=(NBLK,),
        in_specs=[
            pl.BlockSpec((BN, C), lambda i: (i, 0)),
            pl.BlockSpec((C, C), lambda i: (0, 0)),
            pl.BlockSpec((1, C), lambda i: (0, 0)),
        ],
        out_specs=pl.BlockSpec((BN, C), lambda i: (i, 0)),
        out_shape=jax.ShapeDtypeStruct((N, C), jnp.float32),
    )(x, w, b2)


def _wcomp_body(c_ref, b_ref, w_ref):
    w_ref[...] = jnp.dot(c_ref[0], b_ref[0],
                         preferred_element_type=jnp.float32)[None]


def _wcomp(comp, basis2):
    return pl.pallas_call(
        _wcomp_body,
        grid=(L,),
        in_specs=[
            pl.BlockSpec((1, R, NB), lambda i: (i, 0, 0)),
            pl.BlockSpec((1, NB, C * C), lambda i: (i, 0, 0)),
        ],
        out_specs=pl.BlockSpec((1, R, C * C), lambda i: (i, 0, 0)),
        out_shape=jax.ShapeDtypeStruct((L, R, C * C), jnp.float32),
    )(comp, basis2)


def _layer_body(sums_ref, cnt_ref, w_ref, cur_ref, hid_ref, root_ref,
                b_ref, t_ref, cur_o_ref, hid_o_ref, *, li, relu):
    r = pl.program_id(1)

    @pl.when(r == 0)
    def _():
        cur_o_ref[...] = jnp.dot(cur_ref[...], root_ref[...],
                                 preferred_element_type=jnp.float32) + b_ref[...]

    inv = 1.0 / jnp.maximum(cnt_ref[0, :, 0:1] + cnt_ref[1, :, 0:1], 1.0)
    mean = sums_ref[...] * inv
    cur_o_ref[...] += jnp.dot(mean, w_ref[0],
                              preferred_element_type=jnp.float32)

    @pl.when(r == R - 1)
    def _():
        a = cur_o_ref[...]
        if relu:
            a = jnp.maximum(a, 0.0)
        cur_o_ref[...] = a
        hid_o_ref[...] = hid_ref[...] + t_ref[0, li + 1] * a


def _layer(sums, cnt2, w3, cur, hidden, root_i, b2, t2, li, relu):
    body = functools.partial(_layer_body, li=li, relu=relu)
    return pl.pallas_call(
        body,
        grid=(NBLK, R),
        in_specs=[
            pl.BlockSpec((BN, C), lambda i, r: (r * NBLK + i, 0)),
            pl.BlockSpec((2, BN, 16), lambda i, r: (0, r * NBLK + i, 0)),
            pl.BlockSpec((1, C, C), lambda i, r: (r, 0, 0)),
            pl.BlockSpec((BN, C), lambda i, r: (i, 0)),
            pl.BlockSpec((BN, C), lambda i, r: (i, 0)),
            pl.BlockSpec((C, C), lambda i, r: (0, 0)),
            pl.BlockSpec((1, C), lambda i, r: (0, 0)),
            pl.BlockSpec((1, L + 1), lambda i, r: (0, 0)),
        ],
        out_specs=[
            pl.BlockSpec((BN, C), lambda i, r: (i, 0)),
            pl.BlockSpec((BN, C), lambda i, r: (i, 0)),
        ],
        out_shape=[
            jax.ShapeDtypeStruct((N, C), jnp.float32),
            jax.ShapeDtypeStruct((N, C), jnp.float32),
        ],
    )(sums, cnt2, w3, cur, hidden, root_i, b2, t2)


_MESH = plsc.VectorSubcoreMesh(core_axis_name="c", subcore_axis_name="s")


def _counts_body(seg_hbm, out_hbm, ones_v, seg_v, zer_v, acc):
    c = lax.axis_index("c")
    s = lax.axis_index("s")
    for i in range(BATCH):
        ones_v[i] = jnp.full((16,), 1.0, jnp.float32)
        zer_v[i] = jnp.zeros((16,), jnp.float32)
    pltpu.sync_copy(seg_hbm.at[s], seg_v)
    for z in range(STRIPE // BATCH):
        pltpu.sync_copy(zer_v, acc.at[pl.ds(s * STRIPE + z * BATCH, BATCH)])
    plsc.subcore_barrier()
    lo = c * (KB // 2)
    hi = lo + KB // 2

    def body(k, carry):
        pltpu.sync_copy(ones_v, acc.at[seg_v.at[k]], add=True)
        return carry

    lax.fori_loop(lo, hi, body, 0)
    plsc.subcore_barrier()
    pltpu.sync_copy(acc.at[pl.ds(s * STRIPE, STRIPE)],
                    out_hbm.at[c, pl.ds(s * STRIPE, STRIPE)])


@functools.partial(
    pl.kernel,
    out_type=jax.ShapeDtypeStruct((NC, SEG_ROWS, 16), jnp.float32),
    mesh=_MESH,
    compiler_params=pltpu.CompilerParams(use_tc_tiling_on_sc=False),
    scratch_types=[
        pltpu.VMEM((BATCH, 16), jnp.float32),
        pltpu.VMEM((KB, BATCH), jnp.int32),
        pltpu.VMEM((BATCH, 16), jnp.float32),
        pltpu.VMEM_SHARED((SEG_ROWS, 16), jnp.float32),
    ],
)
def _counts(seg_hbm, out_hbm, ones_v, seg_v, zer_v, acc):
    _counts_body(seg_hbm, out_hbm, ones_v, seg_v, zer_v, acc)


N_ST = N // NT


NBUF = 8


def _segsum_body(table, srch, segh, out, zer_v, idx_v, seg_v, rows,
                 gsems, ssems, acc, tbl):
    c = lax.axis_index("c")
    s = lax.axis_index("s")
    for i in range(BATCH):
        zer_v[i] = jnp.zeros((16,), jnp.float32)
    for jj in range(NPASS // NC):
        p = (NPASS // NC) * c + jj
        pltpu.sync_copy(table.at[p, pl.ds(s * N_ST, N_ST)],
                        tbl.at[pl.ds(s * N_ST, N_ST)])
        for z in range(STRIPE // BATCH):
            pltpu.sync_copy(zer_v, acc.at[pl.ds(s * STRIPE + z * BATCH, BATCH)])
        plsc.subcore_barrier()

        for ck in range(KB // KC):
            pltpu.sync_copy(srch.at[s, pl.ds(ck * KC, KC)], idx_v)
            pltpu.sync_copy(segh.at[s, pl.ds(ck * KC, KC)], seg_v)
            for b in range(NBUF):
                pltpu.async_copy(zer_v, acc.at[seg_v.at[0]], ssems[b],
                                 add=True)

            def body(m, carry):
                ds = []
                for b in range(NBUF):
                    k = m * NBUF + b
                    pltpu.make_async_copy(
                        table.at[0, pl.ds(0, BATCH)], rows.at[b],
                        ssems[b]).wait()
                    ds.append(pltpu.async_copy(
                        tbl.at[idx_v.at[k]], rows.at[b], gsems[b]))
                for b in range(NBUF):
                    k = m * NBUF + b
                    ds[b].wait()
                    pltpu.async_copy(rows.at[b], acc.at[seg_v.at[k]],
                                     ssems[b], add=True)
                return carry

            lax.fori_loop(0, KC // NBUF, body, 0)
            for b in range(NBUF):
                pltpu.make_async_copy(
                    table.at[0, pl.ds(0, BATCH)], rows.at[b],
                    ssems[b]).wait()
        plsc.subcore_barrier()
        pltpu.sync_copy(
            acc.at[pl.ds(s * STRIPE, STRIPE)],
            out.at[pl.ds(s * STRIPE, STRIPE), pl.ds(p * 16, 16)])


@functools.partial(
    pl.kernel,
    out_type=jax.ShapeDtypeStruct((SEG_ROWS, C), jnp.float32),
    mesh=_MESH,
    compiler_params=pltpu.CompilerParams(use_tc_tiling_on_sc=False),
    scratch_types=[
        pltpu.VMEM((BATCH, 16), jnp.float32),
        pltpu.VMEM((KC, BATCH), jnp.int32),
        pltpu.VMEM((KC, BATCH), jnp.int32),
        pltpu.VMEM((NBUF, BATCH, 16), jnp.float32),
        [pltpu.SemaphoreType.DMA] * NBUF,
        [pltpu.SemaphoreType.DMA] * NBUF,
        pltpu.VMEM_SHARED((SEG_ROWS, 16), jnp.float32),
        pltpu.VMEM_SHARED((N, 16), jnp.float32),
    ],
)
def _segsum(table, srch, segh, out, zer_v, idx_v, seg_v, rows, gsems,
            ssems, acc, tbl):
    _segsum_body(table, srch, segh, out, zer_v, idx_v, seg_v, rows,
                 gsems, ssems, acc, tbl)


def kernel(x, edge_index, edge_type, temp, lin1_w, lin1_b, lin2_w, lin2_b,
           basis, comp, root, conv_bias):
    src = edge_index[0]
    dst = edge_index[1]
    pad = E_PAD - E
    src2 = jnp.pad(src, (0, pad)).reshape(NT * KB, BATCH)
    dst2 = jnp.pad(dst, (0, pad)).reshape(NT * KB, BATCH)
    et2 = jnp.pad(edge_type, (0, pad), constant_values=R).reshape(
        NT * KB, BATCH)

    seg2 = _prep_idx(dst2, et2)
    srch = src2.reshape(NT, KB, BATCH)
    segh = seg2.reshape(NT, KB, BATCH)

    cnt2 = _counts(segh)

    t2 = temp.reshape(1, L + 1)
    h, hidden = _lin1(x, lin1_w, lin1_b.reshape(1, C), t2)
    wflat = _wcomp(comp, basis.reshape(L, NB, C * C))

    cur = hidden
    for i in range(L):
        table = _totable(cur)
        sums = _segsum(table, srch, segh)
        w3 = wflat[i].reshape(R, C, C)
        cur, hidden = _layer(sums, cnt2, w3, cur, hidden, root[i],
                             conv_bias[i].reshape(1, C), t2, i, i < L - 1)

    return _lin2(hidden, lin2_w, lin2_b.reshape(1, C))


# speedup vs baseline: 1.0487x; 1.0487x over previous
import functools

import jax
import jax.numpy as jnp
from jax import lax
from jax.experimental import pallas as pl
from jax.experimental.pallas import tpu as pltpu
from jax.experimental.pallas import tpu_sc as plsc

N = 10000
E = 320000
C = 128
R = 8
NB = 4
L = 3

NC = 2
NT = 16
BATCH = 128
KB = 160
KC = 40
E_PT = KB * BATCH
E_PAD = NT * E_PT
SEG_ROWS = 81920
STRIPE = SEG_ROWS // NT
NPASS = 8
NBLK = 10
BN = N // NBLK


def _prep_body(dst_ref, et_ref, seg_ref):
    seg_ref[...] = et_ref[...] * N + dst_ref[...]


def _prep_idx(dst2, et2):
    return pl.pallas_call(
        _prep_body,
        out_shape=jax.ShapeDtypeStruct((NT * KB, BATCH), jnp.int32),
    )(dst2, et2)


def _totable_body(x_ref, t_ref):
    for p in range(NPASS):
        t_ref[p] = x_ref[:, 16 * p:16 * p + 16]


def _totable(x):
    return pl.pallas_call(
        _totable_body,
        grid=(NBLK,),
        in_specs=[pl.BlockSpec((BN, C), lambda i: (i, 0))],
        out_specs=pl.BlockSpec((NPASS, BN, 16), lambda i: (0, i, 0)),
        out_shape=jax.ShapeDtypeStruct((NPASS, N, 16), jnp.float32),
    )(x)


def _lin1_body(x_ref, w_ref, b_ref, t_ref, h_ref, hid_ref):
    h = jnp.dot(x_ref[...], w_ref[...], preferred_element_type=jnp.float32)
    h = h + b_ref[...]
    h_ref[...] = h
    hid_ref[...] = t_ref[0, 0] * h


def _lin1(x, w, b2, t2):
    return pl.pallas_call(
        _lin1_body,
        grid=(NBLK,),
        in_specs=[
            pl.BlockSpec((BN, C), lambda i: (i, 0)),
            pl.BlockSpec((C, C), lambda i: (0, 0)),
            pl.BlockSpec((1, C), lambda i: (0, 0)),
            pl.BlockSpec((1, L + 1), lambda i: (0, 0)),
        ],
        out_specs=[
            pl.BlockSpec((BN, C), lambda i: (i, 0)),
            pl.BlockSpec((BN, C), lambda i: (i, 0)),
        ],
        out_shape=[
            jax.ShapeDtypeStruct((N, C), jnp.float32),
            jax.ShapeDtypeStruct((N, C), jnp.float32),
        ],
    )(x, w, b2, t2)


def _lin2_body(x_ref, w_ref, b_ref, o_ref):
    o_ref[...] = jnp.dot(x_ref[...], w_ref[...],
                         preferred_element_type=jnp.float32) + b_ref[...]


def _lin2(x, w, b2):
    return pl.pallas_call(
        _lin2_body,
        grid=(NBLK,),
        in_specs=[
            pl.BlockSpec((BN, C), lambda i: (i, 0)),
            pl.BlockSpec((C, C), lambda i: (0, 0)),
            pl.BlockSpec((1, C), lambda i: (0, 0)),
        ],
        out_specs=pl.BlockSpec((BN, C), lambda i: (i, 0)),
        out_shape=jax.ShapeDtypeStruct((N, C), jnp.float32),
    )(x, w, b2)


def _wcomp_body(c_ref, b_ref, w_ref):
    w_ref[...] = jnp.dot(c_ref[0], b_ref[0],
                         preferred_element_type=jnp.float32)[None]


def _wcomp(comp, basis2):
    return pl.pallas_call(
        _wcomp_body,
        grid=(L,),
        in_specs=[
            pl.BlockSpec((1, R, NB), lambda i: (i, 0, 0)),
            pl.BlockSpec((1, NB, C * C), lambda i: (i, 0, 0)),
        ],
        out_specs=pl.BlockSpec((1, R, C * C), lambda i: (i, 0, 0)),
        out_shape=jax.ShapeDtypeStruct((L, R, C * C), jnp.float32),
    )(comp, basis2)


def _layer_body(sums_ref, cnt_ref, w_ref, cur_ref, hid_ref, root_ref,
                b_ref, t_ref, cur_o_ref, hid_o_ref, *, li, relu):
    r = pl.program_id(1)

    @pl.when(r == 0)
    def _():
        cur_o_ref[...] = jnp.dot(cur_ref[...], root_ref[...],
                                 preferred_element_type=jnp.float32) + b_ref[...]

    inv = 1.0 / jnp.maximum(cnt_ref[0, :, 0:1] + cnt_ref[1, :, 0:1], 1.0)
    mean = sums_ref[...] * inv
    cur_o_ref[...] += jnp.dot(mean, w_ref[0],
                              preferred_element_type=jnp.float32)

    @pl.when(r == R - 1)
    def _():
        a = cur_o_ref[...]
        if relu:
            a = jnp.maximum(a, 0.0)
        cur_o_ref[...] = a
        hid_o_ref[...] = hid_ref[...] + t_ref[0, li + 1] * a


def _layer(sums, cnt2, w3, cur, hidden, root_i, b2, t2, li, relu):
    body = functools.partial(_layer_body, li=li, relu=relu)
    return pl.pallas_call(
        body,
        grid=(NBLK, R),
        in_specs=[
            pl.BlockSpec((BN, C), lambda i, r: (r * NBLK + i, 0)),
            pl.BlockSpec((2, BN, 16), lambda i, r: (0, r * NBLK + i, 0)),
            pl.BlockSpec((1, C, C), lambda i, r: (r, 0, 0)),
            pl.BlockSpec((BN, C), lambda i, r: (i, 0)),
            pl.BlockSpec((BN, C), lambda i, r: (i, 0)),
            pl.BlockSpec((C, C), lambda i, r: (0, 0)),
            pl.BlockSpec((1, C), lambda i, r: (0, 0)),
            pl.BlockSpec((1, L + 1), lambda i, r: (0, 0)),
        ],
        out_specs=[
            pl.BlockSpec((BN, C), lambda i, r: (i, 0)),
            pl.BlockSpec((BN, C), lambda i, r: (i, 0)),
        ],
        out_shape=[
            jax.ShapeDtypeStruct((N, C), jnp.float32),
            jax.ShapeDtypeStruct((N, C), jnp.float32),
        ],
    )(sums, cnt2, w3, cur, hidden, root_i, b2, t2)


_MESH = plsc.VectorSubcoreMesh(core_axis_name="c", subcore_axis_name="s")


def _counts_body(seg_hbm, out_hbm, ones_v, seg_v, zer_v, acc):
    c = lax.axis_index("c")
    s = lax.axis_index("s")
    for i in range(BATCH):
        ones_v[i] = jnp.full((16,), 1.0, jnp.float32)
        zer_v[i] = jnp.zeros((16,), jnp.float32)
    pltpu.sync_copy(seg_hbm.at[s], seg_v)
    for z in range(STRIPE // BATCH):
        pltpu.sync_copy(zer_v, acc.at[pl.ds(s * STRIPE + z * BATCH, BATCH)])
    plsc.subcore_barrier()
    lo = c * (KB // 2)
    hi = lo + KB // 2

    def body(k, carry):
        pltpu.sync_copy(ones_v, acc.at[seg_v.at[k]], add=True)
        return carry

    lax.fori_loop(lo, hi, body, 0)
    plsc.subcore_barrier()
    pltpu.sync_copy(acc.at[pl.ds(s * STRIPE, STRIPE)],
                    out_hbm.at[c, pl.ds(s * STRIPE, STRIPE)])


@functools.partial(
    pl.kernel,
    out_type=jax.ShapeDtypeStruct((NC, SEG_ROWS, 16), jnp.float32),
    mesh=_MESH,
    compiler_params=pltpu.CompilerParams(use_tc_tiling_on_sc=False),
    scratch_types=[
        pltpu.VMEM((BATCH, 16), jnp.float32),
        pltpu.VMEM((KB, BATCH), jnp.int32),
        pltpu.VMEM((BATCH, 16), jnp.float32),
        pltpu.VMEM_SHARED((SEG_ROWS, 16), jnp.float32),
    ],
)
def _counts(seg_hbm, out_hbm, ones_v, seg_v, zer_v, acc):
    _counts_body(seg_hbm, out_hbm, ones_v, seg_v, zer_v, acc)


N_ST = N // NT


NBUF = 8


def _segsum_body(table, srch, segh, out, zer_v, idx_v, seg_v, rows,
                 gsems, ssems, acc, tbl):
    c = lax.axis_index("c")
    s = lax.axis_index("s")
    for i in range(BATCH):
        zer_v[i] = jnp.zeros((16,), jnp.float32)
    for jj in range(NPASS // NC):
        p = (NPASS // NC) * c + jj
        pltpu.sync_copy(table.at[p, pl.ds(s * N_ST, N_ST)],
                        tbl.at[pl.ds(s * N_ST, N_ST)])
        for z in range(STRIPE // BATCH):
            pltpu.sync_copy(zer_v, acc.at[pl.ds(s * STRIPE + z * BATCH, BATCH)])
        plsc.subcore_barrier()

        for ck in range(KB // KC):
            u = ck % 2
            pltpu.sync_copy(srch.at[s, pl.ds(ck * KC, KC)], idx_v.at[u])
            pltpu.sync_copy(segh.at[s, pl.ds(ck * KC, KC)], seg_v.at[u])
            if ck == 0:
                for b in range(NBUF):
                    pltpu.async_copy(zer_v, acc.at[seg_v.at[0, 0]],
                                     ssems[b], add=True)

            def body(m, carry, u=u):
                ds = []
                for b in range(NBUF):
                    k = m * NBUF + b
                    pltpu.make_async_copy(
                        table.at[0, pl.ds(0, BATCH)], rows.at[b],
                        ssems[b]).wait()
                    ds.append(pltpu.async_copy(
                        tbl.at[idx_v.at[u, k]], rows.at[b], gsems[b]))
                for b in range(NBUF):
                    k = m * NBUF + b
                    ds[b].wait()
                    pltpu.async_copy(rows.at[b], acc.at[seg_v.at[u, k]],
                                     ssems[b], add=True)
                return carry

            lax.fori_loop(0, KC // NBUF, body, 0)
        for b in range(NBUF):
            pltpu.make_async_copy(
                table.at[0, pl.ds(0, BATCH)], rows.at[b],
                ssems[b]).wait()
        plsc.subcore_barrier()
        pltpu.sync_copy(
            acc.at[pl.ds(s * STRIPE, STRIPE)],
            out.at[pl.ds(s * STRIPE, STRIPE), pl.ds(p * 16, 16)])


@functools.partial(
    pl.kernel,
    out_type=jax.ShapeDtypeStruct((SEG_ROWS, C), jnp.float32),
    mesh=_MESH,
    compiler_params=pltpu.CompilerParams(use_tc_tiling_on_sc=False),
    scratch_types=[
        pltpu.VMEM((BATCH, 16), jnp.float32),
        pltpu.VMEM((2, KC, BATCH), jnp.int32),
        pltpu.VMEM((2, KC, BATCH), jnp.int32),
        pltpu.VMEM((NBUF, BATCH, 16), jnp.float32),
        [pltpu.SemaphoreType.DMA] * NBUF,
        [pltpu.SemaphoreType.DMA] * NBUF,
        pltpu.VMEM_SHARED((SEG_ROWS, 16), jnp.float32),
        pltpu.VMEM_SHARED((N, 16), jnp.float32),
    ],
)
def _segsum(table, srch, segh, out, zer_v, idx_v, seg_v, rows, gsems,
            ssems, acc, tbl):
    _segsum_body(table, srch, segh, out, zer_v, idx_v, seg_v, rows,
                 gsems, ssems, acc, tbl)


def kernel(x, edge_index, edge_type, temp, lin1_w, lin1_b, lin2_w, lin2_b,
           basis, comp, root, conv_bias):
    src = edge_index[0]
    dst = edge_index[1]
    pad = E_PAD - E
    src2 = jnp.pad(src, (0, pad)).reshape(NT * KB, BATCH)
    dst2 = jnp.pad(dst, (0, pad)).reshape(NT * KB, BATCH)
    et2 = jnp.pad(edge_type, (0, pad), constant_values=R).reshape(
        NT * KB, BATCH)

    seg2 = _prep_idx(dst2, et2)
    srch = src2.reshape(NT, KB, BATCH)
    segh = seg2.reshape(NT, KB, BATCH)

    cnt2 = _counts(segh)

    t2 = temp.reshape(1, L + 1)
    h, hidden = _lin1(x, lin1_w, lin1_b.reshape(1, C), t2)
    wflat = _wcomp(comp, basis.reshape(L, NB, C * C))

    cur = hidden
    for i in range(L):
        table = _totable(cur)
        sums = _segsum(table, srch, segh)
        w3 = wflat[i].reshape(R, C, C)
        cur, hidden = _layer(sums, cnt2, w3, cur, hidden, root[i],
                             conv_bias[i].reshape(1, C), t2, i, i < L - 1)

    return _lin2(hidden, lin2_w, lin2_b.reshape(1, C))


# speedup vs baseline: 1.0765x; 1.0265x over previous
import functools

import jax
import jax.numpy as jnp
from jax import lax
from jax.experimental import pallas as pl
from jax.experimental.pallas import tpu as pltpu
from jax.experimental.pallas import tpu_sc as plsc

N = 10000
E = 320000
C = 128
R = 8
NB = 4
L = 3

NC = 2
NT = 16
BATCH = 128
KB = 160
KC = 40
E_PT = KB * BATCH
E_PAD = NT * E_PT
SEG_ROWS = 81920
STRIPE = SEG_ROWS // NT
NPASS = 8
NBLK = 10
BN = N // NBLK


def _prep_body(dst_ref, et_ref, seg_ref):
    seg_ref[...] = et_ref[...] * N + dst_ref[...]


def _prep_idx(dst2, et2):
    return pl.pallas_call(
        _prep_body,
        out_shape=jax.ShapeDtypeStruct((NT * KB, BATCH), jnp.int32),
    )(dst2, et2)


def _totable_body(x_ref, t_ref):
    for p in range(NPASS):
        t_ref[p] = x_ref[:, 16 * p:16 * p + 16]


def _totable(x):
    return pl.pallas_call(
        _totable_body,
        grid=(NBLK,),
        in_specs=[pl.BlockSpec((BN, C), lambda i: (i, 0))],
        out_specs=pl.BlockSpec((NPASS, BN, 16), lambda i: (0, i, 0)),
        out_shape=jax.ShapeDtypeStruct((NPASS, N, 16), jnp.float32),
    )(x)


def _lin1_body(x_ref, w_ref, b_ref, t_ref, h_ref, hid_ref):
    h = jnp.dot(x_ref[...], w_ref[...], preferred_element_type=jnp.float32)
    h = h + b_ref[...]
    h_ref[...] = h
    hid_ref[...] = t_ref[0, 0] * h


def _lin1(x, w, b2, t2):
    return pl.pallas_call(
        _lin1_body,
        grid=(NBLK,),
        in_specs=[
            pl.BlockSpec((BN, C), lambda i: (i, 0)),
            pl.BlockSpec((C, C), lambda i: (0, 0)),
            pl.BlockSpec((1, C), lambda i: (0, 0)),
            pl.BlockSpec((1, L + 1), lambda i: (0, 0)),
        ],
        out_specs=[
            pl.BlockSpec((BN, C), lambda i: (i, 0)),
            pl.BlockSpec((BN, C), lambda i: (i, 0)),
        ],
        out_shape=[
            jax.ShapeDtypeStruct((N, C), jnp.float32),
            jax.ShapeDtypeStruct((N, C), jnp.float32),
        ],
    )(x, w, b2, t2)


def _lin2_body(x_ref, w_ref, b_ref, o_ref):
    o_ref[...] = jnp.dot(x_ref[...], w_ref[...],
                         preferred_element_type=jnp.float32) + b_ref[...]


def _lin2(x, w, b2):
    return pl.pallas_call(
        _lin2_body,
        grid=(NBLK,),
        in_specs=[
            pl.BlockSpec((BN, C), lambda i: (i, 0)),
            pl.BlockSpec((C, C), lambda i: (0, 0)),
            pl.BlockSpec((1, C), lambda i: (0, 0)),
        ],
        out_specs=pl.BlockSpec((BN, C), lambda i: (i, 0)),
        out_shape=jax.ShapeDtypeStruct((N, C), jnp.float32),
    )(x, w, b2)


def _wcomp_body(c_ref, b_ref, w_ref):
    w_ref[...] = jnp.dot(c_ref[0], b_ref[0],
                         preferred_element_type=jnp.float32)[None]


def _wcomp(comp, basis2):
    return pl.pallas_call(
        _wcomp_body,
        grid=(L,),
        in_specs=[
            pl.BlockSpec((1, R, NB), lambda i: (i, 0, 0)),
            pl.BlockSpec((1, NB, C * C), lambda i: (i, 0, 0)),
        ],
        out_specs=pl.BlockSpec((1, R, C * C), lambda i: (i, 0, 0)),
        out_shape=jax.ShapeDtypeStruct((L, R, C * C), jnp.float32),
    )(comp, basis2)


def _layer_body(sums_ref, cnt_ref, w_ref, cur_ref, hid_ref, root_ref,
                b_ref, t_ref, cur_o_ref, hid_o_ref, *, li, relu):
    r = pl.program_id(1)

    @pl.when(r == 0)
    def _():
        cur_o_ref[...] = jnp.dot(cur_ref[...], root_ref[...],
                                 preferred_element_type=jnp.float32) + b_ref[...]

    inv = 1.0 / jnp.maximum(cnt_ref[0, :, 0:1] + cnt_ref[1, :, 0:1], 1.0)
    mean = sums_ref[...] * inv
    cur_o_ref[...] += jnp.dot(mean, w_ref[0],
                              preferred_element_type=jnp.float32)

    @pl.when(r == R - 1)
    def _():
        a = cur_o_ref[...]
        if relu:
            a = jnp.maximum(a, 0.0)
        cur_o_ref[...] = a
        hid_o_ref[...] = hid_ref[...] + t_ref[0, li + 1] * a


def _layer(sums, cnt2, w3, cur, hidden, root_i, b2, t2, li, relu):
    body = functools.partial(_layer_body, li=li, relu=relu)
    return pl.pallas_call(
        body,
        grid=(NBLK, R),
        in_specs=[
            pl.BlockSpec((BN, C), lambda i, r: (r * NBLK + i, 0)),
            pl.BlockSpec((2, BN, 16), lambda i, r: (0, r * NBLK + i, 0)),
            pl.BlockSpec((1, C, C), lambda i, r: (r, 0, 0)),
            pl.BlockSpec((BN, C), lambda i, r: (i, 0)),
            pl.BlockSpec((BN, C), lambda i, r: (i, 0)),
            pl.BlockSpec((C, C), lambda i, r: (0, 0)),
            pl.BlockSpec((1, C), lambda i, r: (0, 0)),
            pl.BlockSpec((1, L + 1), lambda i, r: (0, 0)),
        ],
        out_specs=[
            pl.BlockSpec((BN, C), lambda i, r: (i, 0)),
            pl.BlockSpec((BN, C), lambda i, r: (i, 0)),
        ],
        out_shape=[
            jax.ShapeDtypeStruct((N, C), jnp.float32),
            jax.ShapeDtypeStruct((N, C), jnp.float32),
        ],
    )(sums, cnt2, w3, cur, hidden, root_i, b2, t2)


_MESH = plsc.VectorSubcoreMesh(core_axis_name="c", subcore_axis_name="s")


def _counts_body(seg_hbm, out_hbm, ones_v, seg_v, zer_v, acc):
    c = lax.axis_index("c")
    s = lax.axis_index("s")
    for i in range(BATCH):
        ones_v[i] = jnp.full((16,), 1.0, jnp.float32)
        zer_v[i] = jnp.zeros((16,), jnp.float32)
    pltpu.sync_copy(seg_hbm.at[s], seg_v)
    for z in range(STRIPE // BATCH):
        pltpu.sync_copy(zer_v, acc.at[pl.ds(s * STRIPE + z * BATCH, BATCH)])
    plsc.subcore_barrier()
    lo = c * (KB // 2)
    hi = lo + KB // 2

    def body(k, carry):
        pltpu.sync_copy(ones_v, acc.at[seg_v.at[k]], add=True)
        return carry

    lax.fori_loop(lo, hi, body, 0)
    plsc.subcore_barrier()
    pltpu.sync_copy(acc.at[pl.ds(s * STRIPE, STRIPE)],
                    out_hbm.at[c, pl.ds(s * STRIPE, STRIPE)])


@functools.partial(
    pl.kernel,
    out_type=jax.ShapeDtypeStruct((NC, SEG_ROWS, 16), jnp.float32),
    mesh=_MESH,
    compiler_params=pltpu.CompilerParams(use_tc_tiling_on_sc=False),
    scratch_types=[
        pltpu.VMEM((BATCH, 16), jnp.float32),
        pltpu.VMEM((KB, BATCH), jnp.int32),
        pltpu.VMEM((BATCH, 16), jnp.float32),
        pltpu.VMEM_SHARED((SEG_ROWS, 16), jnp.float32),
    ],
)
def _counts(seg_hbm, out_hbm, ones_v, seg_v, zer_v, acc):
    _counts_body(seg_hbm, out_hbm, ones_v, seg_v, zer_v, acc)


N_ST = N // NT


NBUF = 8


def _segsum_body(table, srch, segh, out, zer_v, idx_v, seg_v, rows,
                 gsems, ssems, acc, tbl):
    c = lax.axis_index("c")
    s = lax.axis_index("s")
    for i in range(BATCH):
        zer_v[i] = jnp.zeros((16,), jnp.float32)
    for jj in range(NPASS // NC):
        p = (NPASS // NC) * c + jj
        st = pltpu.async_copy(table.at[p, pl.ds(s * N_ST, N_ST)],
                              tbl.at[pl.ds(s * N_ST, N_ST)], gsems[0])
        nz = STRIPE // BATCH
        for z in range(nz):
            pltpu.async_copy(
                zer_v, acc.at[pl.ds(s * STRIPE + z * BATCH, BATCH)],
                ssems[z % NBUF])
        st.wait()
        for b in range(NBUF):
            for _ in range((nz + NBUF - 1 - b) // NBUF):
                pltpu.make_async_copy(
                    zer_v, acc.at[pl.ds(0, BATCH)], ssems[b]).wait()
        plsc.subcore_barrier()

        for ck in range(KB // KC):
            u = ck % 2
            pltpu.sync_copy(srch.at[s, pl.ds(ck * KC, KC)], idx_v.at[u])
            pltpu.sync_copy(segh.at[s, pl.ds(ck * KC, KC)], seg_v.at[u])
            if ck == 0:
                for b in range(NBUF):
                    pltpu.async_copy(zer_v, acc.at[seg_v.at[0, 0]],
                                     ssems[b], add=True)

            def body(m, carry, u=u):
                ds = []
                for b in range(NBUF):
                    k = m * NBUF + b
                    pltpu.make_async_copy(
                        table.at[0, pl.ds(0, BATCH)], rows.at[b],
                        ssems[b]).wait()
                    ds.append(pltpu.async_copy(
                        tbl.at[idx_v.at[u, k]], rows.at[b], gsems[b]))
                for b in range(NBUF):
                    k = m * NBUF + b
                    ds[b].wait()
                    pltpu.async_copy(rows.at[b], acc.at[seg_v.at[u, k]],
                                     ssems[b], add=True)
                return carry

            lax.fori_loop(0, KC // NBUF, body, 0)
        for b in range(NBUF):
            pltpu.make_async_copy(
                table.at[0, pl.ds(0, BATCH)], rows.at[b],
                ssems[b]).wait()
        plsc.subcore_barrier()
        pltpu.sync_copy(
            acc.at[pl.ds(s * STRIPE, STRIPE)],
            out.at[pl.ds(s * STRIPE, STRIPE), pl.ds(p * 16, 16)])


@functools.partial(
    pl.kernel,
    out_type=jax.ShapeDtypeStruct((SEG_ROWS, C), jnp.float32),
    mesh=_MESH,
    compiler_params=pltpu.CompilerParams(use_tc_tiling_on_sc=False),
    scratch_types=[
        pltpu.VMEM((BATCH, 16), jnp.float32),
        pltpu.VMEM((2, KC, BATCH), jnp.int32),
        pltpu.VMEM((2, KC, BATCH), jnp.int32),
        pltpu.VMEM((NBUF, BATCH, 16), jnp.float32),
        [pltpu.SemaphoreType.DMA] * NBUF,
        [pltpu.SemaphoreType.DMA] * NBUF,
        pltpu.VMEM_SHARED((SEG_ROWS, 16), jnp.float32),
        pltpu.VMEM_SHARED((N, 16), jnp.float32),
    ],
)
def _segsum(table, srch, segh, out, zer_v, idx_v, seg_v, rows, gsems,
            ssems, acc, tbl):
    _segsum_body(table, srch, segh, out, zer_v, idx_v, seg_v, rows,
                 gsems, ssems, acc, tbl)


def kernel(x, edge_index, edge_type, temp, lin1_w, lin1_b, lin2_w, lin2_b,
           basis, comp, root, conv_bias):
    src = edge_index[0]
    dst = edge_index[1]
    pad = E_PAD - E
    src2 = jnp.pad(src, (0, pad)).reshape(NT * KB, BATCH)
    dst2 = jnp.pad(dst, (0, pad)).reshape(NT * KB, BATCH)
    et2 = jnp.pad(edge_type, (0, pad), constant_values=R).reshape(
        NT * KB, BATCH)

    seg2 = _prep_idx(dst2, et2)
    srch = src2.reshape(NT, KB, BATCH)
    segh = seg2.reshape(NT, KB, BATCH)

    cnt2 = _counts(segh)

    t2 = temp.reshape(1, L + 1)
    h, hidden = _lin1(x, lin1_w, lin1_b.reshape(1, C), t2)
    wflat = _wcomp(comp, basis.reshape(L, NB, C * C))

    cur = hidden
    for i in range(L):
        table = _totable(cur)
        sums = _segsum(table, srch, segh)
        w3 = wflat[i].reshape(R, C, C)
        cur, hidden = _layer(sums, cnt2, w3, cur, hidden, root[i],
                             conv_bias[i].reshape(1, C), t2, i, i < L - 1)

    return _lin2(hidden, lin2_w, lin2_b.reshape(1, C))
